# baseline (speedup 1.0000x reference)
import functools

import jax
import jax.numpy as jnp
from jax import lax
from jax.experimental import pallas as pl
from jax.experimental.pallas import tpu as pltpu
from jax.experimental.pallas import tpu_sc as plsc

_INFO = plsc.get_sparse_core_info()
_NC = _INFO.num_cores
_NS = _INFO.num_subcores
_L = _INFO.num_lanes
_NW = _NC * _NS


def _make_glove_kernel(B, V, D):
  BW = B // _NW
  NCH = BW // 128
  NG = 128 // _L

  mesh = plsc.VectorSubcoreMesh(core_axis_name="c", subcore_axis_name="s")

  @functools.partial(
      pl.kernel,
      mesh=mesh,
      out_type=jax.ShapeDtypeStruct((B,), jnp.float32),
      compiler_params=pltpu.CompilerParams(
          needs_layout_passes=False, use_tc_tiling_on_sc=False),
      scratch_types=[
          pltpu.VMEM((BW,), jnp.int32),
          pltpu.VMEM((BW,), jnp.int32),
          pltpu.VMEM((BW, D), jnp.float32),
          pltpu.VMEM((BW, D), jnp.float32),
          pltpu.VMEM((V, 1), jnp.float32),
          pltpu.VMEM((V, 1), jnp.float32),
          pltpu.VMEM((BW,), jnp.float32),
          pltpu.SemaphoreType.DMA,
          pltpu.SemaphoreType.DMA,
          pltpu.SemaphoreType.DMA,
      ],
  )
  def glove(center_hbm, context_hbm, cw_hbm, cb_hbm, xw_hbm, xb_hbm,
            out_hbm, idx_c, idx_x, rows_c, rows_x, cb_v, xb_v,
            out_v, sem_c, sem_x, sem_b):
    wid = lax.axis_index("s") * _NC + lax.axis_index("c")
    base = wid * BW

    pltpu.sync_copy(center_hbm.at[pl.ds(base, BW)], idx_c)
    pltpu.sync_copy(context_hbm.at[pl.ds(base, BW)], idx_x)

    copies = []
    for j in range(NCH):
      copies.append(pltpu.async_copy(
          cw_hbm.at[idx_c.at[pl.ds(j * 128, 128)]],
          rows_c.at[pl.ds(j * 128, 128), :], sem_c))
      copies.append(pltpu.async_copy(
          xw_hbm.at[idx_x.at[pl.ds(j * 128, 128)]],
          rows_x.at[pl.ds(j * 128, 128), :], sem_x))
    bias_c = pltpu.async_copy(cb_hbm, cb_v, sem_b)
    bias_x = pltpu.async_copy(xb_hbm, xb_v, sem_b)

    iot = lax.iota(jnp.int32, _L)
    zero = jnp.zeros((_L,), jnp.int32)
    bias_c.wait()
    bias_x.wait()

    for j in range(NCH):
      copies[2 * j].wait()
      copies[2 * j + 1].wait()

      def group(g, _, j=j):
        rowv = (j * 128 + g * _L) + iot
        ci = idx_c[pl.ds(j * 128 + g * _L, _L)]
        xi = idx_x[pl.ds(j * 128 + g * _L, _L)]
        acc = [
            plsc.load_gather(cb_v, [ci, zero]),
            plsc.load_gather(xb_v, [xi, zero]),
            jnp.zeros((_L,), jnp.float32),
            jnp.zeros((_L,), jnp.float32),
        ]
        for c in range(D):
          cf = jnp.full((_L,), c, jnp.int32)
          vc = plsc.load_gather(rows_c, [rowv, cf])
          vx = plsc.load_gather(rows_x, [rowv, cf])
          acc[c % 4] = acc[c % 4] + vc * vx
        out_v[pl.ds(j * 128 + g * _L, _L)] = (
            (acc[0] + acc[1]) + (acc[2] + acc[3]))
        return _

      lax.fori_loop(0, NG, group, 0)

    pltpu.sync_copy(out_v, out_hbm.at[pl.ds(base, BW)])

  return glove


@jax.jit
def kernel(center, context, center_weight, center_bias, context_weight,
           context_bias):
  B = center.shape[0]
  V, D = center_weight.shape
  glove = _make_glove_kernel(B, V, D)
  return glove(center.astype(jnp.int32), context.astype(jnp.int32),
               center_weight, center_bias, context_weight, context_bias)


# speedup vs baseline: 1.0887x; 1.0887x over previous
import functools

import jax
import jax.numpy as jnp
from jax import lax
from jax.experimental import pallas as pl
from jax.experimental.pallas import tpu as pltpu
from jax.experimental.pallas import tpu_sc as plsc

_INFO = plsc.get_sparse_core_info()
_NC = _INFO.num_cores
_NS = _INFO.num_subcores
_L = _INFO.num_lanes
_NW = _NC * _NS


def _make_glove_kernel(B, V, D):
  BW = B // _NW
  NCH = BW // 128
  NG = 128 // _L
  CB = 16
  NCB = D // CB

  mesh = plsc.VectorSubcoreMesh(core_axis_name="c", subcore_axis_name="s")

  @functools.partial(
      pl.kernel,
      mesh=mesh,
      out_type=jax.ShapeDtypeStruct((B,), jnp.float32),
      compiler_params=pltpu.CompilerParams(
          needs_layout_passes=False, use_tc_tiling_on_sc=False),
      scratch_types=[
          pltpu.VMEM((BW,), jnp.int32),
          pltpu.VMEM((BW,), jnp.int32),
          pltpu.VMEM((BW, D), jnp.float32),
          pltpu.VMEM((BW, D), jnp.float32),
          pltpu.VMEM((V, 1), jnp.float32),
          pltpu.VMEM((V, 1), jnp.float32),
          pltpu.VMEM((BW,), jnp.float32),
          pltpu.SemaphoreType.DMA,
          pltpu.SemaphoreType.DMA,
          pltpu.SemaphoreType.DMA,
      ],
  )
  def glove(center_hbm, context_hbm, cw_hbm, cb_hbm, xw_hbm, xb_hbm,
            out_hbm, idx_c, idx_x, rows_c, rows_x, cb_v, xb_v,
            out_v, sem_c, sem_x, sem_b):
    wid = lax.axis_index("s") * _NC + lax.axis_index("c")
    base = wid * BW

    pltpu.sync_copy(center_hbm.at[pl.ds(base, BW)], idx_c)
    pltpu.sync_copy(context_hbm.at[pl.ds(base, BW)], idx_x)

    copies = []
    for j in range(NCH):
      copies.append(pltpu.async_copy(
          cw_hbm.at[idx_c.at[pl.ds(j * 128, 128)]],
          rows_c.at[pl.ds(j * 128, 128), :], sem_c))
      copies.append(pltpu.async_copy(
          xw_hbm.at[idx_x.at[pl.ds(j * 128, 128)]],
          rows_x.at[pl.ds(j * 128, 128), :], sem_x))
    bias_c = pltpu.async_copy(cb_hbm, cb_v, sem_b)
    bias_x = pltpu.async_copy(xb_hbm, xb_v, sem_b)

    iot = lax.iota(jnp.int32, _L)
    zero = jnp.zeros((_L,), jnp.int32)
    bias_c.wait()
    bias_x.wait()

    for j in range(NCH):
      copies[2 * j].wait()
      copies[2 * j + 1].wait()

      def group(g, _, j=j):
        rowv = (j * 128 + g * _L) + iot
        ci = idx_c[pl.ds(j * 128 + g * _L, _L)]
        xi = idx_x[pl.ds(j * 128 + g * _L, _L)]
        bias = (plsc.load_gather(cb_v, [ci, zero])
                + plsc.load_gather(xb_v, [xi, zero]))

        def cblock(blk, accs):
          a0, a1, a2, a3 = accs
          acc = [a0, a1, a2, a3]
          cbase = blk * CB
          for cc in range(CB):
            cf = jnp.full((_L,), cc, jnp.int32) + cbase
            vc = plsc.load_gather(rows_c, [rowv, cf])
            vx = plsc.load_gather(rows_x, [rowv, cf])
            acc[cc % 4] = acc[cc % 4] + vc * vx
          return tuple(acc)

        z = jnp.zeros((_L,), jnp.float32)
        a0, a1, a2, a3 = lax.fori_loop(0, NCB, cblock, (z, z, z, z))
        out_v[pl.ds(j * 128 + g * _L, _L)] = bias + ((a0 + a1) + (a2 + a3))
        return _

      lax.fori_loop(0, NG, group, 0)

    pltpu.sync_copy(out_v, out_hbm.at[pl.ds(base, BW)])

  return glove


@jax.jit
def kernel(center, context, center_weight, center_bias, context_weight,
           context_bias):
  B = center.shape[0]
  V, D = center_weight.shape
  glove = _make_glove_kernel(B, V, D)
  return glove(center.astype(jnp.int32), context.astype(jnp.int32),
               center_weight, center_bias, context_weight, context_bias)


# speedup vs baseline: 1.8474x; 1.6969x over previous
import functools

import jax
import jax.numpy as jnp
from jax import lax
from jax.experimental import pallas as pl
from jax.experimental.pallas import tpu as pltpu
from jax.experimental.pallas import tpu_sc as plsc

_INFO = plsc.get_sparse_core_info()
_NC = _INFO.num_cores
_NS = _INFO.num_subcores
_L = _INFO.num_lanes
_NW = _NC * _NS


def _make_glove_kernel(B, V, D):
  BW = B // _NW
  NCH = BW // 128
  NG = 128 // _L

  mesh = plsc.VectorSubcoreMesh(core_axis_name="c", subcore_axis_name="s")

  @functools.partial(
      pl.kernel,
      mesh=mesh,
      out_type=jax.ShapeDtypeStruct((B,), jnp.float32),
      compiler_params=pltpu.CompilerParams(
          needs_layout_passes=False, use_tc_tiling_on_sc=False),
      scratch_types=[
          pltpu.VMEM((BW,), jnp.int32),
          pltpu.VMEM((BW,), jnp.int32),
          pltpu.VMEM((BW, D), jnp.float32),
          pltpu.VMEM((BW, D), jnp.float32),
          pltpu.VMEM((V, 1), jnp.float32),
          pltpu.VMEM((V, 1), jnp.float32),
          pltpu.VMEM((_L, _L + 1), jnp.float32),
          pltpu.VMEM((BW,), jnp.float32),
          pltpu.SemaphoreType.DMA,
          pltpu.SemaphoreType.DMA,
          pltpu.SemaphoreType.DMA,
      ],
  )
  def glove(center_hbm, context_hbm, cw_hbm, cb_hbm, xw_hbm, xb_hbm,
            out_hbm, idx_c, idx_x, rows_c, rows_x, cb_v, xb_v, tscr,
            out_v, sem_c, sem_x, sem_b):
    wid = lax.axis_index("s") * _NC + lax.axis_index("c")
    base = wid * BW

    pltpu.sync_copy(center_hbm.at[pl.ds(base, BW)], idx_c)
    pltpu.sync_copy(context_hbm.at[pl.ds(base, BW)], idx_x)

    copies = []
    for j in range(NCH):
      copies.append(pltpu.async_copy(
          cw_hbm.at[idx_c.at[pl.ds(j * 128, 128)]],
          rows_c.at[pl.ds(j * 128, 128), :], sem_c))
      copies.append(pltpu.async_copy(
          xw_hbm.at[idx_x.at[pl.ds(j * 128, 128)]],
          rows_x.at[pl.ds(j * 128, 128), :], sem_x))
    bias_c = pltpu.async_copy(cb_hbm, cb_v, sem_b)
    bias_x = pltpu.async_copy(xb_hbm, xb_v, sem_b)

    iot = lax.iota(jnp.int32, _L)
    zero = jnp.zeros((_L,), jnp.int32)
    bias_c.wait()
    bias_x.wait()

    for j in range(NCH):
      copies[2 * j].wait()
      copies[2 * j + 1].wait()

      def group(g, _, j=j):
        rbase = j * 128 + g * _L
        svecs = []
        for i in range(_L):
          row = rbase + i
          s0 = (rows_c[row, pl.ds(0, _L)] * rows_x[row, pl.ds(0, _L)]
                + rows_c[row, pl.ds(_L, _L)] * rows_x[row, pl.ds(_L, _L)])
          s1 = (rows_c[row, pl.ds(2 * _L, _L)] * rows_x[row, pl.ds(2 * _L, _L)]
                + rows_c[row, pl.ds(3 * _L, _L)] * rows_x[row, pl.ds(3 * _L, _L)])
          svecs.append(s0 + s1)
        for i in range(_L):
          tscr[i, pl.ds(0, _L)] = svecs[i]
        ci = idx_c[pl.ds(rbase, _L)]
        xi = idx_x[pl.ds(rbase, _L)]
        acc = (plsc.load_gather(cb_v, [ci, zero])
               + plsc.load_gather(xb_v, [xi, zero]))
        for l in range(_L):
          col = plsc.load_gather(
              tscr, [iot, jnp.full((_L,), l, jnp.int32)])
          acc = acc + col
        out_v[pl.ds(rbase, _L)] = acc
        return _

      lax.fori_loop(0, NG, group, 0)

    pltpu.sync_copy(out_v, out_hbm.at[pl.ds(base, BW)])

  return glove


@jax.jit
def kernel(center, context, center_weight, center_bias, context_weight,
           context_bias):
  B = center.shape[0]
  V, D = center_weight.shape
  glove = _make_glove_kernel(B, V, D)
  return glove(center.astype(jnp.int32), context.astype(jnp.int32),
               center_weight, center_bias, context_weight, context_bias)
